# Initial kernel scaffold; baseline (speedup 1.0000x reference)
import jax
import jax.numpy as jnp
from jax.experimental import pallas as pl


def kernel(x, edge_index, batch, W_l1, b_l1, W_r1, bn1_g, bn1_b, Wp, bp, W_l2, b_l2, W_r2, bn2_g, bn2_b):
    raise NotImplementedError("write your pallas kernel here")


# speedup vs baseline: 16.1349x; 16.1349x over previous
import functools

import jax
import jax.numpy as jnp
from jax import lax
from jax.experimental import pallas as pl
from jax.experimental.pallas import tpu as pltpu
from jax.experimental.pallas import tpu_sc as plsc

N = 10000
E = 320000
DIN = 128
DH = 256
DOUT = 40
G = 64
EPS = 1e-5

NC = 2
NS = 16
NW = NC * NS
EPW = E // NW
CH = 80
NCH = EPW // CH
RPT = 632
RPL = N - (NS - 1) * RPT

D2P = 48
_V16 = 16


@functools.cache
def _make_sc_agg(D, with_hist):
  mesh = plsc.VectorSubcoreMesh(
      core_axis_name="c", subcore_axis_name="s", num_cores=NC,
      num_subcores=NS)
  out_type = [jax.ShapeDtypeStruct((NC * N, D), jnp.float32)]
  scratch = [
      pltpu.VMEM((EPW,), jnp.int32),
      pltpu.VMEM((NCH, CH), jnp.int32),
      pltpu.VMEM((CH, D), jnp.float32),
      pltpu.VMEM((CH, D), jnp.float32),
      pltpu.VMEM_SHARED((N, D), jnp.float32),
      pltpu.SemaphoreType.DMA,
      pltpu.SemaphoreType.DMA,
  ]
  if with_hist:
    out_type.append(jax.ShapeDtypeStruct((NW, N), jnp.float32))
    scratch.append(pltpu.VMEM((N,), jnp.float32))

  @functools.partial(
      pl.kernel,
      out_type=tuple(out_type),
      mesh=mesh,
      scratch_types=scratch,
      compiler_params=pltpu.CompilerParams(
          use_tc_tiling_on_sc=False, needs_layout_passes=False),
  )
  def sc_agg(table, src, dst, zeros, *rest):
    if with_hist:
      out, hout, src_v, dst_v, rows_a, rows_b, acc, sem_a, sem_b, hist = rest
    else:
      out, src_v, dst_v, rows_a, rows_b, acc, sem_a, sem_b = rest
    c = lax.axis_index("c")
    s = lax.axis_index("s")
    wid = c * NS + s
    pltpu.sync_copy(src.at[pl.ds(wid * EPW, EPW)], src_v)
    pltpu.sync_copy(dst.at[wid], dst_v)
    @pl.when(s < NS - 1)
    def _():
      pltpu.sync_copy(zeros.at[pl.ds(s * RPT, RPT)],
                      acc.at[pl.ds(s * RPT, RPT)])

    @pl.when(s == NS - 1)
    def _():
      pltpu.sync_copy(zeros.at[pl.ds((NS - 1) * RPT, RPL)],
                      acc.at[pl.ds((NS - 1) * RPT, RPL)])
    if with_hist:
      zv = jnp.zeros((_V16,), jnp.float32)

      def zbody(i, _):
        hist[pl.ds(i * _V16, _V16)] = zv
        return ()

      lax.fori_loop(0, N // _V16, zbody, (), unroll=False)
    plsc.subcore_barrier()

    ones = jnp.ones((_V16,), jnp.float32)

    def hist_chunk(k):
      if with_hist:
        for q in range(CH // _V16):
          idx = dst_v[k, pl.ds(q * _V16, _V16)]
          plsc.addupdate_scatter(hist, [idx], ones)

    def wait_a():
      pltpu.make_async_copy(table.at[src_v.at[pl.ds(0, CH)]], rows_a,
                            sem_a).wait()

    def wait_b():
      pltpu.make_async_copy(table.at[src_v.at[pl.ds(0, CH)]], rows_b,
                            sem_b).wait()

    pltpu.async_copy(table.at[src_v.at[pl.ds(0, CH)]], rows_a, sem_a)

    def body(g, _):
      k = 2 * g
      pltpu.async_copy(table.at[src_v.at[pl.ds((k + 1) * CH, CH)]], rows_b,
                       sem_b)
      hist_chunk(k)
      wait_a()
      pltpu.sync_copy(rows_a, acc.at[dst_v.at[k]], add=True)
      pltpu.async_copy(table.at[src_v.at[pl.ds((k + 2) * CH, CH)]], rows_a,
                       sem_a)
      hist_chunk(k + 1)
      wait_b()
      pltpu.sync_copy(rows_b, acc.at[dst_v.at[k + 1]], add=True)
      return ()

    lax.fori_loop(0, (NCH - 1) // 2, body, (), unroll=False)
    hist_chunk(NCH - 1)
    wait_a()
    pltpu.sync_copy(rows_a, acc.at[dst_v.at[NCH - 1]], add=True)

    plsc.subcore_barrier()

    @pl.when(s < NS - 1)
    def _():
      pltpu.sync_copy(acc.at[pl.ds(s * RPT, RPT)],
                      out.at[pl.ds(c * N + s * RPT, RPT)])

    @pl.when(s == NS - 1)
    def _():
      pltpu.sync_copy(acc.at[pl.ds((NS - 1) * RPT, RPL)],
                      out.at[pl.ds(c * N + (NS - 1) * RPT, RPL)])
    if with_hist:
      pltpu.sync_copy(hist, hout.at[wid])

  return sc_agg


_DN = (((1,), (1,)), ((), ()))


def _tc_mid_body(sum1p_ref, histp_ref, x_ref, wl1_ref, bl1_ref, wr1_ref,
                 g1_ref, b1_ref, wp_ref, bp_ref, wl2p_ref, wr2_ref, bl2_ref,
                 m_ref, r2_ref, inv_ref):
  sp = sum1p_ref[...]
  s = sp[:N] + sp[N:]
  x = x_ref[...]
  cnt = jnp.sum(histp_ref[...], axis=0)
  inv = 1.0 / jnp.maximum(cnt, 1.0)
  sn = s * inv[:, None]
  t = lax.dot_general(sn, wl1_ref[...], _DN,
                      preferred_element_type=jnp.float32)
  t = t + lax.dot_general(x, wr1_ref[...], _DN,
                          preferred_element_type=jnp.float32)
  t = t + bl1_ref[...]
  mu = jnp.mean(t, axis=0)
  var = jnp.mean((t - mu) ** 2, axis=0)
  h = (t - mu) * lax.rsqrt(var + EPS) * g1_ref[...] + b1_ref[...]
  h = h + lax.dot_general(x, wp_ref[...], _DN,
                          preferred_element_type=jnp.float32) + bp_ref[...]
  h = jnp.maximum(h, 0.0)
  m_ref[...] = lax.dot_general(h, wl2p_ref[...], _DN,
                               preferred_element_type=jnp.float32)
  r2_ref[...] = lax.dot_general(h, wr2_ref[...], _DN,
                                preferred_element_type=jnp.float32) + bl2_ref[...]
  inv_ref[...] = inv[:, None]


def _tc_mid(sum1p, histp, x, wl1, bl1, wr1, g1, b1, wp, bp, wl2p, wr2, bl2):
  return pl.pallas_call(
      _tc_mid_body,
      out_shape=(
          jax.ShapeDtypeStruct((N, D2P), jnp.float32),
          jax.ShapeDtypeStruct((N, DOUT), jnp.float32),
          jax.ShapeDtypeStruct((N, 1), jnp.float32),
      ),
      compiler_params=pltpu.CompilerParams(
          vmem_limit_bytes=100 * 1024 * 1024),
  )(sum1p, histp, x, wl1, bl1, wr1, g1, b1, wp, bp, wl2p, wr2, bl2)


def _tc_out_body(sum2p_ref, inv_ref, r2_ref, batch_ref, g2_ref, b2_ref,
                 out_ref):
  sp = sum2p_ref[...]
  s = (sp[:N] + sp[N:])[:, :DOUT]
  o = s * inv_ref[...] + r2_ref[...]
  mu = jnp.mean(o, axis=0)
  var = jnp.mean((o - mu) ** 2, axis=0)
  o = (o - mu) * lax.rsqrt(var + EPS) * g2_ref[...] + b2_ref[...]
  gid = lax.broadcasted_iota(jnp.int32, (N, G), 1)
  onehot = (batch_ref[...] == gid).astype(jnp.float32)
  ps = lax.dot_general(onehot, o, (((0,), (0,)), ((), ())),
                       preferred_element_type=jnp.float32)
  gc = jnp.sum(onehot, axis=0)
  p = ps / jnp.maximum(gc, 1.0)[:, None]
  mx = jnp.max(p, axis=1, keepdims=True)
  lse = jnp.log(jnp.sum(jnp.exp(p - mx), axis=1, keepdims=True)) + mx
  out_ref[...] = p - lse


def _tc_out(sum2p, inv, r2, batch2d, g2, b2):
  return pl.pallas_call(
      _tc_out_body,
      out_shape=jax.ShapeDtypeStruct((G, DOUT), jnp.float32),
  )(sum2p, inv, r2, batch2d, g2, b2)


def kernel(x, edge_index, batch, W_l1, b_l1, W_r1, bn1_g, bn1_b, Wp, bp,
           W_l2, b_l2, W_r2, bn2_g, bn2_b):
  src = edge_index[0]
  dstr = edge_index[1].reshape(NW, NCH, CH)
  wl2p = jnp.pad(W_l2, ((0, D2P - DOUT), (0, 0)))

  sum1p, histp = _make_sc_agg(DIN, True)(
      x, src, dstr, jnp.zeros((N, DIN), jnp.float32))
  m, r2, inv = _tc_mid(sum1p, histp, x, W_l1, b_l1, W_r1, bn1_g, bn1_b,
                       Wp, bp, wl2p, W_r2, b_l2)
  sum2p, = _make_sc_agg(D2P, False)(
      m, src, dstr, jnp.zeros((N, D2P), jnp.float32))
  return _tc_out(sum2p, inv, r2, batch.reshape(N, 1), bn2_g, bn2_b)


# speedup vs baseline: 17.7244x; 1.0985x over previous
import functools

import jax
import jax.numpy as jnp
from jax import lax
from jax.experimental import pallas as pl
from jax.experimental.pallas import tpu as pltpu
from jax.experimental.pallas import tpu_sc as plsc

N = 10000
E = 320000
DIN = 128
DH = 256
DOUT = 40
G = 64
EPS = 1e-5

NC = 2
NS = 16
NW = NC * NS
CH = 80
EPW = E // NW
NCH = EPW // CH
EPT = E // NS
NCHL = EPT // CH
DH1 = DIN // NC
RPT = 632
RPL = N - (NS - 1) * RPT

D2P = 48
_V16 = 16
NB1 = 6
NB2 = 4


@functools.cache
def _sc_agg_l1():
  mesh = plsc.VectorSubcoreMesh(
      core_axis_name="c", subcore_axis_name="s", num_cores=NC,
      num_subcores=NS)
  scratch = [
      pltpu.VMEM((EPT,), jnp.int32),
      pltpu.VMEM((NCHL, CH), jnp.int32),
  ]
  scratch += [pltpu.VMEM((CH, DH1), jnp.float32) for _ in range(NB1)]
  scratch += [pltpu.VMEM_SHARED((N, DH1), jnp.float32)]
  scratch += [pltpu.SemaphoreType.DMA for _ in range(NB1)]
  scratch += [pltpu.VMEM((N,), jnp.float32)]

  @functools.partial(
      pl.kernel,
      out_type=(jax.ShapeDtypeStruct((NC * N, DH1), jnp.float32),
                jax.ShapeDtypeStruct((NW, N), jnp.float32)),
      mesh=mesh,
      scratch_types=scratch,
      compiler_params=pltpu.CompilerParams(
          use_tc_tiling_on_sc=False, needs_layout_passes=False),
  )
  def sc_agg(table2, src2, dst16, zeros, out, hout, *rest):
    src_v, dst_v = rest[0], rest[1]
    rows = rest[2:2 + NB1]
    acc = rest[2 + NB1]
    sem_g = rest[3 + NB1:3 + 2 * NB1]
    hist = rest[3 + 2 * NB1]
    c = lax.axis_index("c")
    s = lax.axis_index("s")
    wid = c * NS + s
    pltpu.sync_copy(src2.at[pl.ds(c * E + s * EPT, EPT)], src_v)
    pltpu.sync_copy(dst16.at[s], dst_v)

    @pl.when(s < NS - 1)
    def _():
      pltpu.sync_copy(zeros.at[pl.ds(s * RPT, RPT)],
                      acc.at[pl.ds(s * RPT, RPT)])

    @pl.when(s == NS - 1)
    def _():
      pltpu.sync_copy(zeros.at[pl.ds((NS - 1) * RPT, RPL)],
                      acc.at[pl.ds((NS - 1) * RPT, RPL)])

    zv = jnp.zeros((_V16,), jnp.float32)

    def zbody(i, _):
      hist[pl.ds(i * _V16, _V16)] = zv
      return ()

    lax.fori_loop(0, N // _V16, zbody, (), unroll=False)
    plsc.subcore_barrier()

    ones = jnp.ones((_V16,), jnp.float32)

    def hist_chunk(k):
      for q in range(CH // _V16):
        idx = dst_v[k, pl.ds(q * _V16, _V16)]
        plsc.addupdate_scatter(hist, [idx], ones)

    def gather(i, k):
      pltpu.async_copy(table2.at[src_v.at[pl.ds(k * CH, CH)]], rows[i],
                       sem_g[i])

    def wait_g(i):
      pltpu.make_async_copy(table2.at[src_v.at[pl.ds(0, CH)]], rows[i],
                            sem_g[i]).wait()

    for i in range(NB1):
      gather(i, i)

    NG = NCHL // NB1
    NREM = NCHL - NG * NB1

    def body(g, _):
      base = g * NB1
      for i in range(NB1):
        k = base + i
        wait_g(i)
        pltpu.sync_copy(rows[i], acc.at[dst_v.at[k]], add=True)
        hist_chunk(k)

        @pl.when(k + NB1 < NCHL)
        def _():
          gather(i, k + NB1)
      return ()

    lax.fori_loop(0, NG, body, (), unroll=False)
    for i in range(NREM):
      k = NG * NB1 + i
      wait_g(i)
      pltpu.sync_copy(rows[i], acc.at[dst_v.at[k]], add=True)
      hist_chunk(k)

    plsc.subcore_barrier()

    @pl.when(s < NS - 1)
    def _():
      pltpu.sync_copy(acc.at[pl.ds(s * RPT, RPT)],
                      out.at[pl.ds(c * N + s * RPT, RPT)])

    @pl.when(s == NS - 1)
    def _():
      pltpu.sync_copy(acc.at[pl.ds((NS - 1) * RPT, RPL)],
                      out.at[pl.ds(c * N + (NS - 1) * RPT, RPL)])
    pltpu.sync_copy(hist, hout.at[wid])

  return sc_agg


@functools.cache
def _sc_agg_l2():
  mesh = plsc.VectorSubcoreMesh(
      core_axis_name="c", subcore_axis_name="s", num_cores=NC,
      num_subcores=NS)
  scratch = [
      pltpu.VMEM((EPW,), jnp.int32),
      pltpu.VMEM((NCH, CH), jnp.int32),
  ]
  scratch += [pltpu.VMEM((CH, D2P), jnp.float32) for _ in range(NB2)]
  scratch += [pltpu.VMEM_SHARED((N, D2P), jnp.float32)]
  scratch += [pltpu.SemaphoreType.DMA for _ in range(NB2)]

  @functools.partial(
      pl.kernel,
      out_type=jax.ShapeDtypeStruct((NC * N, D2P), jnp.float32),
      mesh=mesh,
      scratch_types=scratch,
      compiler_params=pltpu.CompilerParams(
          use_tc_tiling_on_sc=False, needs_layout_passes=False),
  )
  def sc_agg(table, src, dst, zeros, out, *rest):
    src_v, dst_v = rest[0], rest[1]
    rows = rest[2:2 + NB2]
    acc = rest[2 + NB2]
    sem_g = rest[3 + NB2:3 + 2 * NB2]
    c = lax.axis_index("c")
    s = lax.axis_index("s")
    wid = c * NS + s
    pltpu.sync_copy(src.at[pl.ds(wid * EPW, EPW)], src_v)
    pltpu.sync_copy(dst.at[wid], dst_v)

    @pl.when(s < NS - 1)
    def _():
      pltpu.sync_copy(zeros.at[pl.ds(s * RPT, RPT)],
                      acc.at[pl.ds(s * RPT, RPT)])

    @pl.when(s == NS - 1)
    def _():
      pltpu.sync_copy(zeros.at[pl.ds((NS - 1) * RPT, RPL)],
                      acc.at[pl.ds((NS - 1) * RPT, RPL)])
    plsc.subcore_barrier()

    def gather(i, k):
      pltpu.async_copy(table.at[src_v.at[pl.ds(k * CH, CH)]], rows[i],
                       sem_g[i])

    def wait_g(i):
      pltpu.make_async_copy(table.at[src_v.at[pl.ds(0, CH)]], rows[i],
                            sem_g[i]).wait()

    for i in range(NB2):
      gather(i, i)

    NG = NCH // NB2
    NREM = NCH - NG * NB2

    def body(g, _):
      base = g * NB2
      for i in range(NB2):
        k = base + i
        wait_g(i)
        pltpu.sync_copy(rows[i], acc.at[dst_v.at[k]], add=True)

        @pl.when(k + NB2 < NCH)
        def _():
          gather(i, k + NB2)
      return ()

    lax.fori_loop(0, NG, body, (), unroll=False)
    for i in range(NREM):
      k = NG * NB2 + i
      wait_g(i)
      pltpu.sync_copy(rows[i], acc.at[dst_v.at[k]], add=True)

    plsc.subcore_barrier()

    @pl.when(s < NS - 1)
    def _():
      pltpu.sync_copy(acc.at[pl.ds(s * RPT, RPT)],
                      out.at[pl.ds(c * N + s * RPT, RPT)])

    @pl.when(s == NS - 1)
    def _():
      pltpu.sync_copy(acc.at[pl.ds((NS - 1) * RPT, RPL)],
                      out.at[pl.ds(c * N + (NS - 1) * RPT, RPL)])

  return sc_agg


_DN = (((1,), (1,)), ((), ()))


def _tc_mid_body(sum1p_ref, histp_ref, x_ref, wl1_ref, bl1_ref, wr1_ref,
                 g1_ref, b1_ref, wp_ref, bp_ref, wl2p_ref, wr2_ref, bl2_ref,
                 m_ref, r2_ref, inv_ref):
  sp = sum1p_ref[...]
  s = jnp.concatenate([sp[:N], sp[N:]], axis=1)
  x = x_ref[...]
  cnt = 0.5 * jnp.sum(histp_ref[...], axis=0)
  inv = 1.0 / jnp.maximum(cnt, 1.0)
  sn = s * inv[:, None]
  t = lax.dot_general(sn, wl1_ref[...], _DN,
                      preferred_element_type=jnp.float32)
  t = t + lax.dot_general(x, wr1_ref[...], _DN,
                          preferred_element_type=jnp.float32)
  t = t + bl1_ref[...]
  mu = jnp.mean(t, axis=0)
  var = jnp.mean((t - mu) ** 2, axis=0)
  h = (t - mu) * lax.rsqrt(var + EPS) * g1_ref[...] + b1_ref[...]
  h = h + lax.dot_general(x, wp_ref[...], _DN,
                          preferred_element_type=jnp.float32) + bp_ref[...]
  h = jnp.maximum(h, 0.0)
  m_ref[...] = lax.dot_general(h, wl2p_ref[...], _DN,
                               preferred_element_type=jnp.float32)
  r2_ref[...] = lax.dot_general(h, wr2_ref[...], _DN,
                                preferred_element_type=jnp.float32) + bl2_ref[...]
  inv_ref[...] = inv[:, None]


def _tc_mid(sum1p, histp, x, wl1, bl1, wr1, g1, b1, wp, bp, wl2p, wr2, bl2):
  return pl.pallas_call(
      _tc_mid_body,
      out_shape=(
          jax.ShapeDtypeStruct((N, D2P), jnp.float32),
          jax.ShapeDtypeStruct((N, DOUT), jnp.float32),
          jax.ShapeDtypeStruct((N, 1), jnp.float32),
      ),
      compiler_params=pltpu.CompilerParams(
          vmem_limit_bytes=100 * 1024 * 1024),
  )(sum1p, histp, x, wl1, bl1, wr1, g1, b1, wp, bp, wl2p, wr2, bl2)


def _tc_out_body(sum2p_ref, inv_ref, r2_ref, batch_ref, g2_ref, b2_ref,
                 out_ref):
  sp = sum2p_ref[...]
  s = (sp[:N] + sp[N:])[:, :DOUT]
  o = s * inv_ref[...] + r2_ref[...]
  mu = jnp.mean(o, axis=0)
  var = jnp.mean((o - mu) ** 2, axis=0)
  o = (o - mu) * lax.rsqrt(var + EPS) * g2_ref[...] + b2_ref[...]
  gid = lax.broadcasted_iota(jnp.int32, (N, G), 1)
  onehot = (batch_ref[...] == gid).astype(jnp.float32)
  ps = lax.dot_general(onehot, o, (((0,), (0,)), ((), ())),
                       preferred_element_type=jnp.float32)
  gc = jnp.sum(onehot, axis=0)
  p = ps / jnp.maximum(gc, 1.0)[:, None]
  mx = jnp.max(p, axis=1, keepdims=True)
  lse = jnp.log(jnp.sum(jnp.exp(p - mx), axis=1, keepdims=True)) + mx
  out_ref[...] = p - lse


def _tc_out(sum2p, inv, r2, batch2d, g2, b2):
  return pl.pallas_call(
      _tc_out_body,
      out_shape=jax.ShapeDtypeStruct((G, DOUT), jnp.float32),
  )(sum2p, inv, r2, batch2d, g2, b2)


def kernel(x, edge_index, batch, W_l1, b_l1, W_r1, bn1_g, bn1_b, Wp, bp,
           W_l2, b_l2, W_r2, bn2_g, bn2_b):
  src = edge_index[0]
  dst = edge_index[1]
  dstr16 = dst.reshape(NS, NCHL, CH)
  dstr32 = dst.reshape(NW, NCH, CH)
  table2 = jnp.concatenate([x[:, :DH1], x[:, DH1:]], axis=0)
  src2 = jnp.concatenate([src, src + N])
  wl2p = jnp.pad(W_l2, ((0, D2P - DOUT), (0, 0)))

  sum1p, histp = _sc_agg_l1()(
      table2, src2, dstr16, jnp.zeros((N, DH1), jnp.float32))
  m, r2, inv = _tc_mid(sum1p, histp, x, W_l1, b_l1, W_r1, bn1_g, bn1_b,
                       Wp, bp, wl2p, W_r2, b_l2)
  sum2p = _sc_agg_l2()(
      m, src, dstr32, jnp.zeros((N, D2P), jnp.float32))
  return _tc_out(sum2p, inv, r2, batch.reshape(N, 1), bn2_g, bn2_b)
